# Initial kernel scaffold; baseline (speedup 1.0000x reference)
import jax
import jax.numpy as jnp
from jax.experimental import pallas as pl


def kernel(x, edge_index, W0, b0, W1, b1):
    raise NotImplementedError("write your pallas kernel here")


# speedup vs baseline: 9.3619x; 9.3619x over previous
import functools

import jax
import jax.numpy as jnp
from jax import lax
from jax.experimental import pallas as pl
from jax.experimental.pallas import tpu as pltpu
from jax.experimental.pallas import tpu_sc as plsc

_N = 10000
_E = 320000
_D = 128
_DH = _D // 2
_NP = 10240
_NC = 2
_NS = 16
_NW = _NC * _NS
_B = 125
_NB = _E // _NW // _B
_NB2 = _E // _NS // _B
_TS = _NP // _NS

_mesh = plsc.VectorSubcoreMesh(core_axis_name="c", subcore_axis_name="s")


def _fill_1d(ref, n, value):
    def body(i, carry):
        ref[pl.ds(i * 16, 16)] = jnp.full((16,), value, jnp.float32)
        return carry
    lax.fori_loop(0, n // 16, body, 0)


@functools.partial(
    pl.kernel,
    out_type=jax.ShapeDtypeStruct((_NC, 2, _NP), jnp.float32),
    mesh=_mesh,
    compiler_params=pltpu.CompilerParams(use_tc_tiling_on_sc=False),
    scratch_types=[
        pltpu.VMEM((_NB, _B), jnp.int32),
        pltpu.VMEM((_NB, _B), jnp.int32),
        pltpu.VMEM((128,), jnp.float32),
        pltpu.VMEM((_TS,), jnp.float32),
        pltpu.VMEM_SHARED((_NP,), jnp.float32),
        pltpu.VMEM_SHARED((_NP,), jnp.float32),
    ],
)
def _deg_kernel(src_hbm, dst_hbm, deg_hbm, src_v, dst_v, ones_v, zero_v,
                dego_sh, degi_sh):
    c = lax.axis_index("c")
    s = lax.axis_index("s")
    w = c * _NS + s
    pltpu.sync_copy(src_hbm.at[w], src_v)
    pltpu.sync_copy(dst_hbm.at[w], dst_v)
    _fill_1d(ones_v, 128, 1.0)
    _fill_1d(zero_v, _TS, 0.0)
    pltpu.sync_copy(zero_v, dego_sh.at[pl.ds(s * _TS, _TS)])
    pltpu.sync_copy(zero_v, degi_sh.at[pl.ds(s * _TS, _TS)])
    plsc.subcore_barrier()

    def body(j, carry):
        pltpu.sync_copy(ones_v.at[pl.ds(0, _B)], dego_sh.at[src_v.at[j]],
                        add=True)
        pltpu.sync_copy(ones_v.at[pl.ds(0, _B)], degi_sh.at[dst_v.at[j]],
                        add=True)
        return carry
    lax.fori_loop(0, _NB, body, 0)
    plsc.subcore_barrier()
    sl = pl.ds(s * _TS, _TS)
    pltpu.sync_copy(dego_sh.at[sl], deg_hbm.at[c, 0, sl])
    pltpu.sync_copy(degi_sh.at[sl], deg_hbm.at[c, 1, sl])


def _prep_body(x_ref, deg_ref, xs_ref, ns_ref, nd_ref):
    deg = deg_ref[0] + deg_ref[1]
    norm = lax.rsqrt(jnp.maximum(deg, 1.0))
    ns = norm[0]
    nd = norm[1]
    ns_ref[...] = ns
    nd_ref[...] = nd
    x = x_ref[...]
    mask = (jnp.sum(x, axis=1, keepdims=True) != 0.0).astype(jnp.float32)
    xs_ref[...] = x * (mask * ns[:_N])


@functools.partial(
    pl.kernel,
    out_type=(
        jax.ShapeDtypeStruct((_NC, _NP, _DH), jnp.float32),
        jax.ShapeDtypeStruct((_NC, _NP), jnp.float32),
    ),
    mesh=_mesh,
    compiler_params=pltpu.CompilerParams(use_tc_tiling_on_sc=False),
    scratch_types=[
        pltpu.VMEM((_NB2, _B), jnp.int32),
        pltpu.VMEM((_NB, _B), jnp.int32),
        pltpu.VMEM((_NB2, _B), jnp.int32),
        pltpu.VMEM((_B, _DH), jnp.float32),
        pltpu.VMEM((_B,), jnp.float32),
        pltpu.VMEM((128, _DH), jnp.float32),
        pltpu.VMEM((_TS,), jnp.float32),
        pltpu.VMEM_SHARED((_NP, _DH), jnp.float32),
        pltpu.VMEM_SHARED((_NP,), jnp.float32),
        pltpu.SemaphoreType.DMA,
        pltpu.SemaphoreType.DMA,
    ],
)
def _edge_kernel(xsr_hbm, nd_hbm, srcx_hbm, srcp_hbm, dst_hbm, agg_hbm, t_hbm,
                 srcx_v, srcp_v, dst_v, rows_v, nvals_v, zero_v, zt_v,
                 agg_sh, t_sh, sem_r, sem_n):
    c = lax.axis_index("c")
    s = lax.axis_index("s")
    pltpu.sync_copy(srcx_hbm.at[c, s], srcx_v)
    pltpu.sync_copy(srcp_hbm.at[s, pl.ds(c * _NB, _NB)], srcp_v)
    pltpu.sync_copy(dst_hbm.at[s], dst_v)

    def zf(i, carry):
        for k in range(_DH // 16):
            zero_v[i, pl.ds(k * 16, 16)] = jnp.zeros((16,), jnp.float32)
        return carry
    lax.fori_loop(0, 128, zf, 0)
    _fill_1d(zt_v, _TS, 0.0)
    for q in range(_TS // 128):
        pltpu.sync_copy(zero_v, agg_sh.at[pl.ds(s * _TS + q * 128, 128), :])
    pltpu.sync_copy(zt_v, t_sh.at[pl.ds(s * _TS, _TS)])
    plsc.subcore_barrier()

    def body(j, carry):
        pltpu.async_copy(xsr_hbm.at[srcx_v.at[j]], rows_v, sem_r).wait()
        pltpu.sync_copy(rows_v, agg_sh.at[dst_v.at[j]], add=True)
        return carry
    lax.fori_loop(0, _NB2, body, 0)

    def tbody(jj, carry):
        pltpu.async_copy(nd_hbm.at[dst_v.at[c * _NB + jj]], nvals_v,
                         sem_n).wait()
        pltpu.sync_copy(nvals_v, t_sh.at[srcp_v.at[jj]], add=True)
        return carry
    lax.fori_loop(0, _NB, tbody, 0)
    plsc.subcore_barrier()
    sl = pl.ds(s * _TS, _TS)
    pltpu.sync_copy(agg_sh.at[sl, :], agg_hbm.at[c, sl, :])
    pltpu.sync_copy(t_sh.at[sl], t_hbm.at[c, sl])


_RB = 1280


def _final_body(agg_ref, t_ref, ns_ref, nd_ref, w0_ref, b0_ref, w1_ref,
                b1_ref, out_ref, acc_ref):
    i = pl.program_id(0)

    @pl.when(i == 0)
    def _init():
        acc_ref[...] = jnp.zeros_like(acc_ref)

    a = jnp.concatenate([agg_ref[0], agg_ref[1]], axis=1)
    pre = a * nd_ref[...]
    z = jnp.dot(pre, w0_ref[...], preferred_element_type=jnp.float32)
    h = jnp.maximum(z + b0_ref[...], 0.0)
    cvec = ns_ref[...] * (t_ref[0] + t_ref[1])
    acc_ref[...] += jnp.sum(h * cvec, axis=0, keepdims=True)

    @pl.when(i == pl.num_programs(0) - 1)
    def _fin():
        v = acc_ref[...] * (1.0 / _N)
        out_ref[...] = (
            jnp.dot(v, w1_ref[...], preferred_element_type=jnp.float32)
            + b1_ref[...]
        )


def kernel(x, edge_index, W0, b0, W1, b1):
    src = edge_index[0]
    dst = edge_index[1]
    src3 = src.reshape(_NW, _NB, _B)
    dst3 = dst.reshape(_NW, _NB, _B)

    deg = _deg_kernel(src3, dst3)

    xs, ns, nd = pl.pallas_call(
        _prep_body,
        out_shape=(
            jax.ShapeDtypeStruct((_N, _D), jnp.float32),
            jax.ShapeDtypeStruct((_NP, 1), jnp.float32),
            jax.ShapeDtypeStruct((_NP, 1), jnp.float32),
        ),
    )(x, deg.reshape(_NC, 2, _NP, 1))

    src2 = src * 2
    srcx = jnp.stack([src2, src2 + 1]).reshape(_NC, _NS, _NB2, _B)
    srcp = src.reshape(_NS, _NB2, _B)
    dst2 = dst.reshape(_NS, _NB2, _B)
    agg, t = _edge_kernel(xs.reshape(2 * _N, _DH), nd.reshape(_NP),
                          srcx, srcp, dst2)

    out = pl.pallas_call(
        _final_body,
        grid=(_NP // _RB,),
        in_specs=[
            pl.BlockSpec((_NC, _RB, _DH), lambda i: (0, i, 0)),
            pl.BlockSpec((_NC, _RB, 1), lambda i: (0, i, 0)),
            pl.BlockSpec((_RB, 1), lambda i: (i, 0)),
            pl.BlockSpec((_RB, 1), lambda i: (i, 0)),
            pl.BlockSpec((_D, _D), lambda i: (0, 0)),
            pl.BlockSpec((1, _D), lambda i: (0, 0)),
            pl.BlockSpec((_D, _D), lambda i: (0, 0)),
            pl.BlockSpec((1, _D), lambda i: (0, 0)),
        ],
        out_specs=pl.BlockSpec((1, _D), lambda i: (0, 0)),
        out_shape=jax.ShapeDtypeStruct((1, _D), jnp.float32),
        scratch_shapes=[pltpu.VMEM((1, _D), jnp.float32)],
    )(agg, t.reshape(_NC, _NP, 1), ns, nd,
      W0, b0.reshape(1, _D), W1, b1.reshape(1, _D))
    return out


# speedup vs baseline: 15.9765x; 1.7065x over previous
import functools

import jax
import jax.numpy as jnp
from jax import lax
from jax.experimental import pallas as pl
from jax.experimental.pallas import tpu as pltpu
from jax.experimental.pallas import tpu_sc as plsc

_N = 10000
_E = 320000
_D = 128
_DH = _D // 2
_NP = 10240
_NC = 2
_NS = 16
_NW = _NC * _NS
_B = 125
_NB = _E // _NW // _B
_NB2 = _E // _NS // _B
_TS = _NP // _NS

_mesh = plsc.VectorSubcoreMesh(core_axis_name="c", subcore_axis_name="s")


def _fill_1d(ref, n, value):
    def body(i, carry):
        ref[pl.ds(i * 16, 16)] = jnp.full((16,), value, jnp.float32)
        return carry
    lax.fori_loop(0, n // 16, body, 0)


@functools.partial(
    pl.kernel,
    out_type=jax.ShapeDtypeStruct((_NC, 2, _NP), jnp.float32),
    mesh=_mesh,
    compiler_params=pltpu.CompilerParams(use_tc_tiling_on_sc=False),
    scratch_types=[
        pltpu.VMEM((_NB, _B), jnp.int32),
        pltpu.VMEM((_NB, _B), jnp.int32),
        pltpu.VMEM((128,), jnp.float32),
        pltpu.VMEM((_TS,), jnp.float32),
        pltpu.VMEM_SHARED((_NP,), jnp.float32),
        pltpu.VMEM_SHARED((_NP,), jnp.float32),
    ],
)
def _deg_kernel(src_hbm, dst_hbm, deg_hbm, src_v, dst_v, ones_v, zero_v,
                dego_sh, degi_sh):
    c = lax.axis_index("c")
    s = lax.axis_index("s")
    w = c * _NS + s
    pltpu.sync_copy(src_hbm.at[w], src_v)
    pltpu.sync_copy(dst_hbm.at[w], dst_v)
    _fill_1d(ones_v, 128, 1.0)
    _fill_1d(zero_v, _TS, 0.0)
    pltpu.sync_copy(zero_v, dego_sh.at[pl.ds(s * _TS, _TS)])
    pltpu.sync_copy(zero_v, degi_sh.at[pl.ds(s * _TS, _TS)])
    plsc.subcore_barrier()

    def body(j, carry):
        pltpu.sync_copy(ones_v.at[pl.ds(0, _B)], dego_sh.at[src_v.at[j]],
                        add=True)
        pltpu.sync_copy(ones_v.at[pl.ds(0, _B)], degi_sh.at[dst_v.at[j]],
                        add=True)
        return carry
    lax.fori_loop(0, _NB, body, 0)
    plsc.subcore_barrier()
    sl = pl.ds(s * _TS, _TS)
    pltpu.sync_copy(dego_sh.at[sl], deg_hbm.at[c, 0, sl])
    pltpu.sync_copy(degi_sh.at[sl], deg_hbm.at[c, 1, sl])


def _prep_body(x_ref, deg_ref, xs_ref, ns_ref, nd_ref):
    deg = deg_ref[0] + deg_ref[1]
    norm = lax.rsqrt(jnp.maximum(deg, 1.0))
    ns = norm[0]
    nd = norm[1]
    ns_ref[...] = ns
    nd_ref[...] = nd
    x = x_ref[...]
    mask = (jnp.sum(x, axis=1, keepdims=True) != 0.0).astype(jnp.float32)
    xs_ref[...] = x * (mask * ns[:_N])


_RING = 4


@functools.partial(
    pl.kernel,
    out_type=(
        jax.ShapeDtypeStruct((_NC, _NP, _DH), jnp.float32),
        jax.ShapeDtypeStruct((_NC, _NP), jnp.float32),
    ),
    mesh=_mesh,
    compiler_params=pltpu.CompilerParams(use_tc_tiling_on_sc=False),
    scratch_types=[
        pltpu.VMEM((_NB2 + _RING, _B), jnp.int32),
        pltpu.VMEM((_NB, _B), jnp.int32),
        pltpu.VMEM((_NB2 + 2, _B), jnp.int32),
        pltpu.VMEM((_B, _DH), jnp.float32),
        pltpu.VMEM((_B, _DH), jnp.float32),
        pltpu.VMEM((_B, _DH), jnp.float32),
        pltpu.VMEM((_B, _DH), jnp.float32),
        pltpu.VMEM((_B,), jnp.float32),
        pltpu.VMEM((_B,), jnp.float32),
        pltpu.VMEM((64, _DH), jnp.float32),
        pltpu.VMEM((_TS,), jnp.float32),
        pltpu.VMEM_SHARED((_NP, _DH), jnp.float32),
        pltpu.VMEM_SHARED((_NP,), jnp.float32),
        pltpu.SemaphoreType.DMA,
        pltpu.SemaphoreType.DMA,
        pltpu.SemaphoreType.DMA,
        pltpu.SemaphoreType.DMA,
        pltpu.SemaphoreType.DMA,
        pltpu.SemaphoreType.DMA,
    ],
)
def _edge_kernel(xsr_hbm, nd_hbm, srcx_hbm, srcp_hbm, dst_hbm, agg_hbm, t_hbm,
                 srcx_v, srcp_v, dst_v, rows0, rows1, rows2, rows3,
                 nv0, nv1, zero_v, zt_v, agg_sh, t_sh,
                 semr0, semr1, semr2, semr3, semn0, semn1):
    c = lax.axis_index("c")
    s = lax.axis_index("s")
    rows = (rows0, rows1, rows2, rows3)
    semr = (semr0, semr1, semr2, semr3)
    nvals = (nv0, nv1)
    semn = (semn0, semn1)
    pltpu.sync_copy(srcx_hbm.at[c, s], srcx_v)
    pltpu.sync_copy(srcp_hbm.at[s, pl.ds(c * _NB, _NB)], srcp_v)
    pltpu.sync_copy(dst_hbm.at[s], dst_v)

    for u in range(_RING):
        pltpu.async_copy(xsr_hbm.at[srcx_v.at[u]], rows[u], semr[u])
    for v in range(2):
        pltpu.async_copy(nd_hbm.at[dst_v.at[c * _NB + v]], nvals[v], semn[v])

    def zf(i, carry):
        for k in range(_DH // 16):
            zero_v[i, pl.ds(k * 16, 16)] = jnp.zeros((16,), jnp.float32)
        return carry
    lax.fori_loop(0, 64, zf, 0)
    _fill_1d(zt_v, _TS, 0.0)
    for q in range(_TS // 64):
        pltpu.sync_copy(zero_v, agg_sh.at[pl.ds(s * _TS + q * 64, 64), :])
    pltpu.sync_copy(zt_v, t_sh.at[pl.ds(s * _TS, _TS)])
    plsc.subcore_barrier()

    def body(q, carry):
        for u in range(_RING):
            j = q * _RING + u
            pltpu.make_async_copy(xsr_hbm.at[srcx_v.at[j]], rows[u],
                                  semr[u]).wait()
            pltpu.sync_copy(rows[u], agg_sh.at[dst_v.at[j]], add=True)
            pltpu.async_copy(xsr_hbm.at[srcx_v.at[j + _RING]], rows[u],
                             semr[u])
            if u % 2 == 1:
                v = u // 2
                jj = q * 2 + v
                pltpu.make_async_copy(nd_hbm.at[dst_v.at[c * _NB + jj]],
                                      nvals[v], semn[v]).wait()
                pltpu.sync_copy(nvals[v], t_sh.at[srcp_v.at[jj]], add=True)
                pltpu.async_copy(nd_hbm.at[dst_v.at[c * _NB + jj + 2]],
                                 nvals[v], semn[v])
        return carry
    lax.fori_loop(0, _NB2 // _RING, body, 0)

    for u in range(_RING):
        pltpu.make_async_copy(xsr_hbm.at[srcx_v.at[_NB2]], rows[u],
                              semr[u]).wait()
    for v in range(2):
        pltpu.make_async_copy(nd_hbm.at[dst_v.at[c * _NB]], nvals[v],
                              semn[v]).wait()
    plsc.subcore_barrier()
    sl = pl.ds(s * _TS, _TS)
    pltpu.sync_copy(agg_sh.at[sl, :], agg_hbm.at[c, sl, :])
    pltpu.sync_copy(t_sh.at[sl], t_hbm.at[c, sl])


_RB = 1280


def _final_body(agg_ref, t_ref, ns_ref, nd_ref, w0_ref, b0_ref, w1_ref,
                b1_ref, out_ref, acc_ref):
    i = pl.program_id(0)

    @pl.when(i == 0)
    def _init():
        acc_ref[...] = jnp.zeros_like(acc_ref)

    a = jnp.concatenate([agg_ref[0], agg_ref[1]], axis=1)
    pre = a * nd_ref[...]
    z = jnp.dot(pre, w0_ref[...], preferred_element_type=jnp.float32)
    h = jnp.maximum(z + b0_ref[...], 0.0)
    cvec = ns_ref[...] * (t_ref[0] + t_ref[1])
    acc_ref[...] += jnp.sum(h * cvec, axis=0, keepdims=True)

    @pl.when(i == pl.num_programs(0) - 1)
    def _fin():
        v = acc_ref[...] * (1.0 / _N)
        out_ref[...] = (
            jnp.dot(v, w1_ref[...], preferred_element_type=jnp.float32)
            + b1_ref[...]
        )


def kernel(x, edge_index, W0, b0, W1, b1):
    src = edge_index[0]
    dst = edge_index[1]
    src3 = src.reshape(_NW, _NB, _B)
    dst3 = dst.reshape(_NW, _NB, _B)

    deg = _deg_kernel(src3, dst3)

    xs, ns, nd = pl.pallas_call(
        _prep_body,
        out_shape=(
            jax.ShapeDtypeStruct((_N, _D), jnp.float32),
            jax.ShapeDtypeStruct((_NP, 1), jnp.float32),
            jax.ShapeDtypeStruct((_NP, 1), jnp.float32),
        ),
    )(x, deg.reshape(_NC, 2, _NP, 1))

    src2 = src * 2
    srcx = jnp.stack([src2, src2 + 1]).reshape(_NC, _NS, _NB2, _B)
    srcp = src.reshape(_NS, _NB2, _B)
    dst2 = dst.reshape(_NS, _NB2, _B)
    srcx = jnp.concatenate([srcx, srcx[:, :, :_RING]], axis=2)
    dst2p = jnp.concatenate([dst2, dst2[:, :2]], axis=1)
    agg, t = _edge_kernel(xs.reshape(2 * _N, _DH), nd.reshape(_NP),
                          srcx, srcp, dst2p)

    out = pl.pallas_call(
        _final_body,
        grid=(_NP // _RB,),
        in_specs=[
            pl.BlockSpec((_NC, _RB, _DH), lambda i: (0, i, 0)),
            pl.BlockSpec((_NC, _RB, 1), lambda i: (0, i, 0)),
            pl.BlockSpec((_RB, 1), lambda i: (i, 0)),
            pl.BlockSpec((_RB, 1), lambda i: (i, 0)),
            pl.BlockSpec((_D, _D), lambda i: (0, 0)),
            pl.BlockSpec((1, _D), lambda i: (0, 0)),
            pl.BlockSpec((_D, _D), lambda i: (0, 0)),
            pl.BlockSpec((1, _D), lambda i: (0, 0)),
        ],
        out_specs=pl.BlockSpec((1, _D), lambda i: (0, 0)),
        out_shape=jax.ShapeDtypeStruct((1, _D), jnp.float32),
        scratch_shapes=[pltpu.VMEM((1, _D), jnp.float32)],
    )(agg, t.reshape(_NC, _NP, 1), ns, nd,
      W0, b0.reshape(1, _D), W1, b1.reshape(1, _D))
    return out
